# baseline (speedup 1.0000x reference)
import functools

import jax
import jax.numpy as jnp
from jax import lax
from jax.experimental import pallas as pl
from jax.experimental.pallas import tpu as pltpu
from jax.experimental.pallas import tpu_sc as plsc

_DIM = 64
_SCALE = 8.0
_LANES = 16
_NC = 2
_NS = 16
_NW = _NC * _NS
_CHUNK = 128
_TCBN = 4096


def _tc_pack(table_t):
    dim, vocab = table_t.shape
    nblk = (vocab + _TCBN - 1) // _TCBN
    rows = nblk * (_TCBN // 2)

    def body(x_ref, out_ref):
        x = x_ref[...]
        r = lax.broadcasted_iota(jnp.int32, (dim, dim), 0)
        c = lax.broadcasted_iota(jnp.int32, (dim, dim), 1)
        ident8 = jnp.where(r == c, _SCALE, 0.0).astype(jnp.float32)
        dn = (((0,), (0,)), ((), ()))
        a = x[:, : _TCBN // 2]
        b = x[:, _TCBN // 2 :]
        ta = lax.dot_general(a, ident8, dn,
                             preferred_element_type=jnp.float32)
        tb = lax.dot_general(b, ident8, dn,
                             preferred_element_type=jnp.float32)
        out_ref[...] = jnp.concatenate([ta, tb], axis=1)

    return pl.pallas_call(
        body,
        grid=(nblk,),
        in_specs=[pl.BlockSpec((dim, _TCBN), lambda i: (0, i))],
        out_specs=pl.BlockSpec((_TCBN // 2, 2 * dim), lambda i: (i, 0)),
        out_shape=jax.ShapeDtypeStruct((rows, 2 * dim), jnp.float32),
    )(table_t)


def _sc_embed(hi_t, table3):
    seq, batch = hi_t.shape
    kpl = batch // _CHUNK
    nchunks = seq * kpl
    cpw = nchunks // _NW
    mesh = plsc.VectorSubcoreMesh(core_axis_name="c", subcore_axis_name="s")

    @functools.partial(
        pl.kernel,
        mesh=mesh,
        out_type=jax.ShapeDtypeStruct(
            (seq, _DIM // 8, batch // _CHUNK, 8, _CHUNK), jnp.float32
        ),
        compiler_params=pltpu.CompilerParams(
            use_tc_tiling_on_sc=False, needs_layout_passes=False
        ),
        scratch_types=[
            pltpu.VMEM((_CHUNK,), jnp.int32),
            pltpu.VMEM((_CHUNK,), jnp.int32),
            pltpu.VMEM((_CHUNK,), jnp.int32),
            pltpu.VMEM((_CHUNK,), jnp.int32),
            pltpu.VMEM((_CHUNK, _DIM), jnp.float32),
            pltpu.VMEM((_CHUNK, _DIM), jnp.float32),
            pltpu.VMEM((_DIM // 8, 8, _CHUNK + 1), jnp.float32),
            pltpu.VMEM((_DIM // 8, 8, _CHUNK + 1), jnp.float32),
            pltpu.SemaphoreType.DMA,
            pltpu.SemaphoreType.DMA,
            pltpu.SemaphoreType.DMA,
            pltpu.SemaphoreType.DMA,
            pltpu.SemaphoreType.DMA,
            pltpu.SemaphoreType.DMA,
            pltpu.SemaphoreType.DMA,
            pltpu.SemaphoreType.DMA,
        ],
    )
    def body(hi_hbm, table_hbm, out_hbm, h0, h1, h2, h3, g0, g1, t0, t1,
             hs0, hs1, hs2, hs3, gs0, gs1, ss0, ss1):
        wid = lax.axis_index("s") * _NC + lax.axis_index("c")
        cbase = wid * cpw

        hbufs = (h0, h1, h2, h3)
        hsems = (hs0, hs1, hs2, hs3)
        gbufs = (g0, g1)
        gsems = (gs0, gs1)
        tbufs = (t0, t1)
        ssems = (ss0, ss1)
        iot = lax.iota(jnp.int32, _LANES)

        def lk(j):
            c = cbase + j
            return c // kpl, lax.rem(c, kpl)

        def hi_start(j, hslot):
            l, k = lk(j)
            pltpu.async_copy(hi_hbm.at[l, pl.ds(k * _CHUNK, _CHUNK)],
                             hbufs[hslot], hsems[hslot])

        def hi_wait(hslot):
            pltpu.make_async_copy(hi_hbm.at[0, pl.ds(0, _CHUNK)],
                                  hbufs[hslot], hsems[hslot]).wait()

        def gather_start(hslot, gslot):
            pltpu.async_copy(table_hbm.at[hbufs[hslot]], gbufs[gslot],
                             gsems[gslot])

        def gather_wait(gslot):
            pltpu.make_async_copy(table_hbm.at[hbufs[0]], gbufs[gslot],
                                  gsems[gslot]).wait()

        def store_start(j, gslot):
            l, k = lk(j)
            pltpu.async_copy(tbufs[gslot].at[:, :, pl.ds(0, _CHUNK)],
                             out_hbm.at[l, :, k], ssems[gslot])

        def store_wait(gslot):
            pltpu.make_async_copy(tbufs[gslot].at[:, :, pl.ds(0, _CHUNK)],
                                  out_hbm.at[0, :, 0],
                                  ssems[gslot]).wait()

        def transpose_scale(gslot):
            g = gbufs[gslot]
            t = tbufs[gslot]

            @plsc.parallel_loop(0, _CHUNK, unroll=8)
            def _(k):
                ks = iot * 0 + k
                for c in range(_DIM // _LANES):
                    f16 = iot + (c * _LANES)
                    trs = lax.shift_right_logical(f16, 3)
                    frs = f16 & 7
                    v = g[k, pl.ds(c * _LANES, _LANES)]
                    plsc.store_scatter(t, [trs, frs, ks], v)

        for j in range(4):
            hi_start(j, j)
        hi_wait(0)
        gather_start(0, 0)
        hi_wait(1)
        gather_start(1, 1)

        def step(tt, carry):
            for u in range(4):
                j = 4 * tt + u
                gslot = u % 2

                @pl.when(j >= 2)
                def _():
                    store_wait(gslot)

                gather_wait(gslot)
                transpose_scale(gslot)

                @pl.when(j + 4 < cpw)
                def _():
                    hi_start(j + 4, u)

                store_start(j, gslot)

                @pl.when(j + 2 < cpw)
                def _():
                    hi_wait((u + 2) % 4)
                    gather_start((u + 2) % 4, gslot)
            return carry

        lax.fori_loop(0, cpw // 4, step, 0)
        store_wait(0)
        store_wait(1)

    return body(hi_t, table3)


def kernel(x, table):
    b, l = x.shape
    idx_t = x.T.astype(jnp.int32)
    hi_t = (
        ((idx_t >> 12) << 12)
        + ((idx_t & 2047) << 1)
        + ((idx_t >> 11) & 1)
    )
    table2 = _tc_pack(table.T)
    table3 = table2.reshape(2 * table2.shape[0], _DIM)
    out5 = _sc_embed(hi_t, table3)
    outp = jnp.transpose(out5, (2, 4, 0, 1, 3))
    return outp.reshape(b, l, _DIM)


# speedup vs baseline: 1.0033x; 1.0033x over previous
import functools

import jax
import jax.numpy as jnp
from jax import lax
from jax.experimental import pallas as pl
from jax.experimental.pallas import tpu as pltpu
from jax.experimental.pallas import tpu_sc as plsc

_DIM = 64
_SCALE = 8.0
_LANES = 16
_NC = 2
_NS = 16
_NW = _NC * _NS
_CHUNK = 128
_TCBN = 4096


def _tc_pack(table_t):
    dim, vocab = table_t.shape
    nblk = (vocab + _TCBN - 1) // _TCBN
    rows = nblk * (_TCBN // 2)

    def body(x_ref, out_ref):
        x = x_ref[...]
        a = x[:, : _TCBN // 2]
        b = x[:, _TCBN // 2 :]
        out_ref[...] = jnp.concatenate([a.T, b.T], axis=1) * _SCALE

    return pl.pallas_call(
        body,
        grid=(nblk,),
        in_specs=[pl.BlockSpec((dim, _TCBN), lambda i: (0, i))],
        out_specs=pl.BlockSpec((_TCBN // 2, 2 * dim), lambda i: (i, 0)),
        out_shape=jax.ShapeDtypeStruct((rows, 2 * dim), jnp.float32),
    )(table_t)


def _sc_embed(hi_t, table3):
    seq, batch = hi_t.shape
    kpl = batch // _CHUNK
    nchunks = seq * kpl
    cpw = nchunks // _NW
    mesh = plsc.VectorSubcoreMesh(core_axis_name="c", subcore_axis_name="s")

    @functools.partial(
        pl.kernel,
        mesh=mesh,
        out_type=jax.ShapeDtypeStruct(
            (seq, _DIM // 8, batch // _CHUNK, 8, _CHUNK), jnp.float32
        ),
        compiler_params=pltpu.CompilerParams(
            use_tc_tiling_on_sc=False, needs_layout_passes=False
        ),
        scratch_types=[
            pltpu.VMEM((_CHUNK,), jnp.int32),
            pltpu.VMEM((_CHUNK,), jnp.int32),
            pltpu.VMEM((_CHUNK,), jnp.int32),
            pltpu.VMEM((_CHUNK,), jnp.int32),
            pltpu.VMEM((_CHUNK, _DIM), jnp.float32),
            pltpu.VMEM((_CHUNK, _DIM), jnp.float32),
            pltpu.VMEM((_DIM // 8, 8, _CHUNK + 1), jnp.float32),
            pltpu.VMEM((_DIM // 8, 8, _CHUNK + 1), jnp.float32),
            pltpu.SemaphoreType.DMA,
            pltpu.SemaphoreType.DMA,
            pltpu.SemaphoreType.DMA,
            pltpu.SemaphoreType.DMA,
            pltpu.SemaphoreType.DMA,
            pltpu.SemaphoreType.DMA,
            pltpu.SemaphoreType.DMA,
            pltpu.SemaphoreType.DMA,
        ],
    )
    def body(hi_hbm, table_hbm, out_hbm, h0, h1, h2, h3, g0, g1, t0, t1,
             hs0, hs1, hs2, hs3, gs0, gs1, ss0, ss1):
        wid = lax.axis_index("s") * _NC + lax.axis_index("c")
        cbase = wid * cpw

        hbufs = (h0, h1, h2, h3)
        hsems = (hs0, hs1, hs2, hs3)
        gbufs = (g0, g1)
        gsems = (gs0, gs1)
        tbufs = (t0, t1)
        ssems = (ss0, ss1)
        iot = lax.iota(jnp.int32, _LANES)

        def lk(j):
            c = cbase + j
            return c // kpl, lax.rem(c, kpl)

        def hi_start(j, hslot):
            l, k = lk(j)
            pltpu.async_copy(hi_hbm.at[l, pl.ds(k * _CHUNK, _CHUNK)],
                             hbufs[hslot], hsems[hslot])

        def hi_wait(hslot):
            pltpu.make_async_copy(hi_hbm.at[0, pl.ds(0, _CHUNK)],
                                  hbufs[hslot], hsems[hslot]).wait()

        def gather_start(hslot, gslot):
            pltpu.async_copy(table_hbm.at[hbufs[hslot]], gbufs[gslot],
                             gsems[gslot])

        def gather_wait(gslot):
            pltpu.make_async_copy(table_hbm.at[hbufs[0]], gbufs[gslot],
                                  gsems[gslot]).wait()

        def store_start(j, gslot):
            l, k = lk(j)
            pltpu.async_copy(tbufs[gslot].at[:, :, pl.ds(0, _CHUNK)],
                             out_hbm.at[l, :, k], ssems[gslot])

        def store_wait(gslot):
            pltpu.make_async_copy(tbufs[gslot].at[:, :, pl.ds(0, _CHUNK)],
                                  out_hbm.at[0, :, 0],
                                  ssems[gslot]).wait()

        def transpose_scale(gslot):
            g = gbufs[gslot]
            t = tbufs[gslot]

            @plsc.parallel_loop(0, _CHUNK, unroll=8)
            def _(k):
                ks = iot * 0 + k
                for c in range(_DIM // _LANES):
                    f16 = iot + (c * _LANES)
                    trs = lax.shift_right_logical(f16, 3)
                    frs = f16 & 7
                    v = g[k, pl.ds(c * _LANES, _LANES)]
                    plsc.store_scatter(t, [trs, frs, ks], v)

        for j in range(4):
            hi_start(j, j)
        hi_wait(0)
        gather_start(0, 0)
        hi_wait(1)
        gather_start(1, 1)

        def step(tt, carry):
            for u in range(4):
                j = 4 * tt + u
                gslot = u % 2

                @pl.when(j >= 2)
                def _():
                    store_wait(gslot)

                gather_wait(gslot)
                transpose_scale(gslot)

                @pl.when(j + 4 < cpw)
                def _():
                    hi_start(j + 4, u)

                store_start(j, gslot)

                @pl.when(j + 2 < cpw)
                def _():
                    hi_wait((u + 2) % 4)
                    gather_start((u + 2) % 4, gslot)
            return carry

        lax.fori_loop(0, cpw // 4, step, 0)
        store_wait(0)
        store_wait(1)

    return body(hi_t, table3)


def kernel(x, table):
    b, l = x.shape
    idx_t = x.T.astype(jnp.int32)
    hi_t = (
        ((idx_t >> 12) << 12)
        + ((idx_t & 2047) << 1)
        + ((idx_t >> 11) & 1)
    )
    table2 = _tc_pack(table.T)
    table3 = table2.reshape(2 * table2.shape[0], _DIM)
    out5 = _sc_embed(hi_t, table3)
    outp = jnp.transpose(out5, (2, 4, 0, 1, 3))
    return outp.reshape(b, l, _DIM)
